# baseline (speedup 1.0000x reference)
import functools

import jax
import jax.numpy as jnp
import numpy as np
from jax import lax
from jax.experimental import pallas as pl
from jax.experimental.pallas import tpu as pltpu
from jax.experimental.pallas import tpu_sc as plsc

N_ELEMS = 8388608
N_BINS = 15
NC = 2
NS = 16
NW = NC * NS
L = 16

SC_K = 10
SC_ELEMS = N_ELEMS * SC_K // 16
TC_ELEMS = N_ELEMS - SC_ELEMS

PER_W = SC_ELEMS // NW
CHUNK = 16384
NCHUNK = PER_W // CHUNK
VECS = CHUNK // L

TC_COLS = 512
TC_BR = 256
TC_STEP = TC_BR * TC_COLS
TC_ROW0 = SC_ELEMS // TC_COLS
TC_STEPS = TC_ELEMS // TC_STEP

_BOUNDS_F32 = np.linspace(0.0, 1.0, N_BINS + 1).astype(np.float32)
_BTAB = np.tile(_BOUNDS_F32, L)


@functools.cache
def _build_kernels():
  mesh = plsc.VectorSubcoreMesh(
      core_axis_name="c", subcore_axis_name="s", num_cores=NC, num_subcores=NS
  )

  @functools.partial(
      pl.kernel,
      out_type=jax.ShapeDtypeStruct((NW, 48), jnp.float32),
      mesh=mesh,
      scratch_types=[
          pltpu.VMEM((2, CHUNK), jnp.float32),
          pltpu.VMEM((2, CHUNK), jnp.int32),
          pltpu.VMEM((L * L,), jnp.float32),
          pltpu.VMEM((L * L,), jnp.int32),
          pltpu.VMEM((L * L,), jnp.float32),
          pltpu.VMEM((48,), jnp.float32),
          pltpu.SemaphoreType.DMA,
          pltpu.SemaphoreType.DMA,
          pltpu.SemaphoreType.DMA,
          pltpu.SemaphoreType.DMA,
      ],
      compiler_params=pltpu.CompilerParams(needs_layout_passes=False),
  )
  def _hist_kernel(conf_hbm, tgt_hbm, btab_hbm, out_hbm,
                   conf_v, tgt_v, btab_v, pk_acc, conf_acc, stage_v,
                   sc0, st0, sc1, st1):
    wid = lax.axis_index("s") * NC + lax.axis_index("c")
    base = wid * PER_W

    pltpu.sync_copy(btab_hbm, btab_v)

    zero_i = jnp.zeros((L,), jnp.int32)
    zero_f = jnp.zeros((L,), jnp.float32)
    for r in range(L):
      pk_acc[pl.ds(r * L, L)] = zero_i
      conf_acc[pl.ds(r * L, L)] = zero_f

    sems = ((sc0, st0), (sc1, st1))

    def start_dma(c):
      b = c % 2
      hc = pltpu.async_copy(
          conf_hbm.at[pl.ds(base + c * CHUNK, CHUNK)], conf_v.at[b], sems[b][0])
      ht = pltpu.async_copy(
          tgt_hbm.at[pl.ds(base + c * CHUNK, CHUNK)], tgt_v.at[b], sems[b][1])
      return hc, ht

    handles = {0: start_dma(0), 1: start_dma(1)}

    lane16 = lax.iota(jnp.int32, L) * L

    for c in range(NCHUNK):
      b = c % 2
      hc, ht = handles.pop(c)
      hc.wait()
      ht.wait()

      @plsc.parallel_loop(0, CHUNK, step=L, unroll=8)
      def _chunk_body(o, b=b):
        cf = conf_v[b, pl.ds(o, L)]
        tg = tgt_v[b, pl.ds(o, L)]
        m = (cf * 15.0 + 0.5).astype(jnp.int32)
        tm = lane16 + m
        bm = plsc.load_gather(btab_v, [tm])
        j = m - (cf <= bm).astype(jnp.int32)
        valid = j >= 0
        pred = (cf >= 0.5).astype(jnp.int32)
        wrong = pred ^ tg
        pk = 32769 - (wrong << 15)
        sidx = lane16 + j
        plsc.addupdate_scatter(pk_acc, [sidx], pk, mask=valid)
        plsc.addupdate_scatter(conf_acc, [sidx], cf, mask=valid)

      if c + 2 < NCHUNK:
        handles[c + 2] = start_dma(c + 2)

    conf_tot = jnp.zeros((L,), jnp.float32)
    cnt_tot = jnp.zeros((L,), jnp.float32)
    acc_tot = jnp.zeros((L,), jnp.float32)
    for r in range(L):
      conf_tot = conf_tot + conf_acc[pl.ds(r * L, L)]
      row = pk_acc[pl.ds(r * L, L)]
      cnt_tot = cnt_tot + (row & 32767).astype(jnp.float32)
      acc_tot = acc_tot + (row >> 15).astype(jnp.float32)

    stage_v[pl.ds(0, L)] = cnt_tot
    stage_v[pl.ds(L, L)] = conf_tot
    stage_v[pl.ds(2 * L, L)] = acc_tot
    pltpu.sync_copy(stage_v, out_hbm.at[wid])

  def _tc_hist_body(conf_ref, tgt_ref, out_ref):
    g = pl.program_id(0)

    @pl.when(g == 0)
    def _():
      for i in range(48):
        out_ref[i] = jnp.float32(0.0)

    c = conf_ref[...]
    t = tgt_ref[...]
    accur = ((c >= 0.5) == (t == 1)).astype(jnp.float32)
    for i in range(N_BINS):
      m = c <= float(_BOUNDS_F32[i])
      out_ref[i] += jnp.sum(m.astype(jnp.float32))
      out_ref[16 + i] += jnp.sum(jnp.where(m, c, 0.0))
      out_ref[32 + i] += jnp.sum(jnp.where(m, accur, 0.0))
    out_ref[31] += jnp.sum(c)
    out_ref[47] += jnp.sum(accur)

  def _tc_hist_kernel(conf, tgt):
    blk0 = SC_ELEMS // TC_STEP
    return pl.pallas_call(
        _tc_hist_body,
        grid=(TC_STEPS,),
        in_specs=[
            pl.BlockSpec((TC_STEP,), lambda g: (blk0 + g,)),
            pl.BlockSpec((TC_STEP,), lambda g: (blk0 + g,)),
        ],
        out_specs=pl.BlockSpec(memory_space=pltpu.SMEM),
        out_shape=jax.ShapeDtypeStruct((48,), jnp.float32),
    )(conf, tgt)

  def _finalize_body(part_ref, tcv_ref, out_ref):
    s = jnp.sum(part_ref[...], axis=0)
    tc = tcv_ref[...]
    def bins(le15, total):
      hi = jnp.concatenate([le15[1:], total[None]])
      return hi - le15
    tc_cnt = bins(tc[0:15], jnp.float32(TC_ELEMS))
    tc_conf = bins(tc[16:31], tc[31])
    tc_acc = bins(tc[32:47], tc[47])
    cnt = s[0:15] + tc_cnt
    conf = s[16:31] + tc_conf
    acc = s[32:47] + tc_acc
    safe = jnp.maximum(cnt, 1.0)
    contrib = jnp.abs(conf / safe - acc / safe) * (cnt * (1.0 / N_ELEMS))
    contrib = jnp.where(cnt > 0.0, contrib, 0.0)
    out_ref[...] = jnp.full((1,), jnp.sum(contrib), jnp.float32)

  def _finalize_kernel(partials, tcvec):
    return pl.pallas_call(
        _finalize_body,
        out_shape=jax.ShapeDtypeStruct((1,), jnp.float32),
    )(partials, tcvec)

  return _hist_kernel, _tc_hist_kernel, _finalize_kernel


def kernel(inputs, targets):
  hist_kernel, tc_hist_kernel, finalize_kernel = _build_kernels()
  btab = jnp.asarray(_BTAB)
  partials = hist_kernel(inputs, targets, btab)
  tcvec = tc_hist_kernel(inputs, targets)
  return finalize_kernel(partials, tcvec)


# speedup vs baseline: 3.4954x; 3.4954x over previous
import functools

import jax
import jax.numpy as jnp
import numpy as np
from jax import lax
from jax.experimental import pallas as pl
from jax.experimental.pallas import tpu as pltpu
from jax.experimental.pallas import tpu_sc as plsc

N_ELEMS = 8388608
N_BINS = 15
NC = 2
NS = 16
NW = NC * NS
L = 16

SC_K = 12
SC_ELEMS = N_ELEMS * SC_K // 16
TC_ELEMS = N_ELEMS - SC_ELEMS

PER_W = SC_ELEMS // NW
CHUNK = 16384
NCHUNK = PER_W // CHUNK
VECS = CHUNK // L

TC_COLS = 512
TC_BR = 256
TC_STEP = TC_BR * TC_COLS
TC_ROW0 = SC_ELEMS // TC_COLS
TC_STEPS = TC_ELEMS // TC_STEP

_BOUNDS_F32 = np.linspace(0.0, 1.0, N_BINS + 1).astype(np.float32)
_BTAB = np.tile(_BOUNDS_F32, L)


@functools.cache
def _build_kernels():
  mesh = plsc.VectorSubcoreMesh(
      core_axis_name="c", subcore_axis_name="s", num_cores=NC, num_subcores=NS
  )

  @functools.partial(
      pl.kernel,
      out_type=jax.ShapeDtypeStruct((NW, 48), jnp.float32),
      mesh=mesh,
      scratch_types=[
          pltpu.VMEM((2, CHUNK), jnp.float32),
          pltpu.VMEM((2, CHUNK), jnp.int32),
          pltpu.VMEM((L * L,), jnp.float32),
          pltpu.VMEM((L * L,), jnp.int32),
          pltpu.VMEM((L * L,), jnp.float32),
          pltpu.VMEM((48,), jnp.float32),
          pltpu.SemaphoreType.DMA,
          pltpu.SemaphoreType.DMA,
          pltpu.SemaphoreType.DMA,
          pltpu.SemaphoreType.DMA,
      ],
      compiler_params=pltpu.CompilerParams(needs_layout_passes=False),
  )
  def _hist_kernel(conf_hbm, tgt_hbm, btab_hbm, out_hbm,
                   conf_v, tgt_v, btab_v, pk_acc, conf_acc, stage_v,
                   sc0, st0, sc1, st1):
    wid = lax.axis_index("s") * NC + lax.axis_index("c")
    base = wid * PER_W

    pltpu.sync_copy(btab_hbm, btab_v)

    zero_i = jnp.zeros((L,), jnp.int32)
    zero_f = jnp.zeros((L,), jnp.float32)
    for r in range(L):
      pk_acc[pl.ds(r * L, L)] = zero_i
      conf_acc[pl.ds(r * L, L)] = zero_f

    sems = ((sc0, st0), (sc1, st1))

    def start_dma(c):
      b = c % 2
      hc = pltpu.async_copy(
          conf_hbm.at[pl.ds(base + c * CHUNK, CHUNK)], conf_v.at[b], sems[b][0])
      ht = pltpu.async_copy(
          tgt_hbm.at[pl.ds(base + c * CHUNK, CHUNK)], tgt_v.at[b], sems[b][1])
      return hc, ht

    handles = {0: start_dma(0), 1: start_dma(1)}

    lane16 = lax.iota(jnp.int32, L) * L

    for c in range(NCHUNK):
      b = c % 2
      hc, ht = handles.pop(c)
      hc.wait()
      ht.wait()

      @plsc.parallel_loop(0, CHUNK, step=L, unroll=8)
      def _chunk_body(o, b=b):
        cf = conf_v[b, pl.ds(o, L)]
        tg = tgt_v[b, pl.ds(o, L)]
        m = (cf * 15.0 + 0.5).astype(jnp.int32)
        tm = lane16 + m
        bm = plsc.load_gather(btab_v, [tm])
        j = m - (cf <= bm).astype(jnp.int32)
        valid = j >= 0
        pred = (cf >= 0.5).astype(jnp.int32)
        wrong = pred ^ tg
        pk = 32769 - (wrong << 15)
        sidx = lane16 + j
        plsc.addupdate_scatter(pk_acc, [sidx], pk, mask=valid)
        plsc.addupdate_scatter(conf_acc, [sidx], cf, mask=valid)

      if c + 2 < NCHUNK:
        handles[c + 2] = start_dma(c + 2)

    conf_tot = jnp.zeros((L,), jnp.float32)
    cnt_tot = jnp.zeros((L,), jnp.float32)
    acc_tot = jnp.zeros((L,), jnp.float32)
    for r in range(L):
      conf_tot = conf_tot + conf_acc[pl.ds(r * L, L)]
      row = pk_acc[pl.ds(r * L, L)]
      cnt_tot = cnt_tot + (row & 32767).astype(jnp.float32)
      acc_tot = acc_tot + (row >> 15).astype(jnp.float32)

    stage_v[pl.ds(0, L)] = cnt_tot
    stage_v[pl.ds(L, L)] = conf_tot
    stage_v[pl.ds(2 * L, L)] = acc_tot
    pltpu.sync_copy(stage_v, out_hbm.at[wid])

  def _tc_hist_body(conf_ref, tgt_ref, out_ref):
    g = pl.program_id(0)

    @pl.when(g == 0)
    def _():
      for i in range(48):
        out_ref[i] = jnp.float32(0.0)

    c = conf_ref[...]
    t = tgt_ref[...]
    accur = ((c >= 0.5) == (t == 1)).astype(jnp.float32)
    for i in range(N_BINS):
      m = c <= float(_BOUNDS_F32[i])
      out_ref[i] += jnp.sum(m.astype(jnp.float32))
      out_ref[16 + i] += jnp.sum(jnp.where(m, c, 0.0))
      out_ref[32 + i] += jnp.sum(jnp.where(m, accur, 0.0))
    out_ref[31] += jnp.sum(c)
    out_ref[47] += jnp.sum(accur)

  def _tc_hist_kernel(conf2d, tgt2d):
    return pl.pallas_call(
        _tc_hist_body,
        grid=(TC_STEPS,),
        in_specs=[
            pl.BlockSpec((TC_BR, TC_COLS), lambda g: (g, 0)),
            pl.BlockSpec((TC_BR, TC_COLS), lambda g: (g, 0)),
        ],
        out_specs=pl.BlockSpec(memory_space=pltpu.SMEM),
        out_shape=jax.ShapeDtypeStruct((48,), jnp.float32),
    )(conf2d, tgt2d)

  def _finalize_body(part_ref, tcv_ref, out_ref):
    s = jnp.sum(part_ref[...], axis=0)
    tc = tcv_ref[...]
    def bins(le15, total):
      hi = jnp.concatenate([le15[1:], total[None]])
      return hi - le15
    tc_cnt = bins(tc[0:15], jnp.float32(TC_ELEMS))
    tc_conf = bins(tc[16:31], tc[31])
    tc_acc = bins(tc[32:47], tc[47])
    cnt = s[0:15] + tc_cnt
    conf = s[16:31] + tc_conf
    acc = s[32:47] + tc_acc
    safe = jnp.maximum(cnt, 1.0)
    contrib = jnp.abs(conf / safe - acc / safe) * (cnt * (1.0 / N_ELEMS))
    contrib = jnp.where(cnt > 0.0, contrib, 0.0)
    out_ref[...] = jnp.full((1,), jnp.sum(contrib), jnp.float32)

  def _finalize_kernel(partials, tcvec):
    return pl.pallas_call(
        _finalize_body,
        out_shape=jax.ShapeDtypeStruct((1,), jnp.float32),
    )(partials, tcvec)

  return _hist_kernel, _tc_hist_kernel, _finalize_kernel


def kernel(inputs, targets):
  hist_kernel, tc_hist_kernel, finalize_kernel = _build_kernels()
  btab = jnp.asarray(_BTAB)
  partials = hist_kernel(inputs, targets, btab)
  conf2d = inputs[SC_ELEMS:].reshape(TC_ELEMS // TC_COLS, TC_COLS)
  tgt2d = targets[SC_ELEMS:].reshape(TC_ELEMS // TC_COLS, TC_COLS)
  tcvec = tc_hist_kernel(conf2d, tgt2d)
  return finalize_kernel(partials, tcvec)


# speedup vs baseline: 5.8668x; 1.6784x over previous
import functools

import jax
import jax.numpy as jnp
import numpy as np
from jax import lax
from jax.experimental import pallas as pl
from jax.experimental.pallas import tpu as pltpu
from jax.experimental.pallas import tpu_sc as plsc

N_ELEMS = 8388608
N_BINS = 15
NC = 2
NS = 16
NW = NC * NS
L = 16

SC_K = 12
SC_ELEMS = N_ELEMS * SC_K // 16
TC_ELEMS = N_ELEMS - SC_ELEMS

PER_W = SC_ELEMS // NW
CHUNK = 16384
NCHUNK = PER_W // CHUNK
VECS = CHUNK // L

TC_COLS = 128
TC_BR = 2048
TC_STEP = TC_BR * TC_COLS
TC_ROW0 = SC_ELEMS // TC_COLS
TC_STEPS = TC_ELEMS // TC_STEP

_BOUNDS_F32 = np.linspace(0.0, 1.0, N_BINS + 1).astype(np.float32)
_BTAB = np.tile(_BOUNDS_F32, L)


@functools.cache
def _build_kernels():
  mesh = plsc.VectorSubcoreMesh(
      core_axis_name="c", subcore_axis_name="s", num_cores=NC, num_subcores=NS
  )

  @functools.partial(
      pl.kernel,
      out_type=jax.ShapeDtypeStruct((NW, 48), jnp.float32),
      mesh=mesh,
      scratch_types=[
          pltpu.VMEM((2, CHUNK), jnp.float32),
          pltpu.VMEM((2, CHUNK), jnp.int32),
          pltpu.VMEM((L * L,), jnp.float32),
          pltpu.VMEM((L * L,), jnp.int32),
          pltpu.VMEM((L * L,), jnp.float32),
          pltpu.VMEM((48,), jnp.float32),
          pltpu.SemaphoreType.DMA,
          pltpu.SemaphoreType.DMA,
          pltpu.SemaphoreType.DMA,
          pltpu.SemaphoreType.DMA,
      ],
      compiler_params=pltpu.CompilerParams(needs_layout_passes=False),
  )
  def _hist_kernel(conf_hbm, tgt_hbm, btab_hbm, out_hbm,
                   conf_v, tgt_v, btab_v, pk_acc, conf_acc, stage_v,
                   sc0, st0, sc1, st1):
    wid = lax.axis_index("s") * NC + lax.axis_index("c")
    base = wid * PER_W

    pltpu.sync_copy(btab_hbm, btab_v)

    zero_i = jnp.zeros((L,), jnp.int32)
    zero_f = jnp.zeros((L,), jnp.float32)
    for r in range(L):
      pk_acc[pl.ds(r * L, L)] = zero_i
      conf_acc[pl.ds(r * L, L)] = zero_f

    sems = ((sc0, st0), (sc1, st1))

    def start_dma(c):
      b = c % 2
      hc = pltpu.async_copy(
          conf_hbm.at[pl.ds(base + c * CHUNK, CHUNK)], conf_v.at[b], sems[b][0])
      ht = pltpu.async_copy(
          tgt_hbm.at[pl.ds(base + c * CHUNK, CHUNK)], tgt_v.at[b], sems[b][1])
      return hc, ht

    handles = {0: start_dma(0), 1: start_dma(1)}

    lane16 = lax.iota(jnp.int32, L) * L

    for c in range(NCHUNK):
      b = c % 2
      hc, ht = handles.pop(c)
      hc.wait()
      ht.wait()

      @plsc.parallel_loop(0, CHUNK, step=L, unroll=8)
      def _chunk_body(o, b=b):
        cf = conf_v[b, pl.ds(o, L)]
        tg = tgt_v[b, pl.ds(o, L)]
        m = (cf * 15.0 + 0.5).astype(jnp.int32)
        tm = lane16 + m
        bm = plsc.load_gather(btab_v, [tm])
        j = m - (cf <= bm).astype(jnp.int32)
        valid = j >= 0
        pred = (cf >= 0.5).astype(jnp.int32)
        wrong = pred ^ tg
        pk = 32769 - (wrong << 15)
        sidx = lane16 + j
        plsc.addupdate_scatter(pk_acc, [sidx], pk, mask=valid)
        plsc.addupdate_scatter(conf_acc, [sidx], cf, mask=valid)

      if c + 2 < NCHUNK:
        handles[c + 2] = start_dma(c + 2)

    conf_tot = jnp.zeros((L,), jnp.float32)
    cnt_tot = jnp.zeros((L,), jnp.float32)
    acc_tot = jnp.zeros((L,), jnp.float32)
    for r in range(L):
      conf_tot = conf_tot + conf_acc[pl.ds(r * L, L)]
      row = pk_acc[pl.ds(r * L, L)]
      cnt_tot = cnt_tot + (row & 32767).astype(jnp.float32)
      acc_tot = acc_tot + (row >> 15).astype(jnp.float32)

    stage_v[pl.ds(0, L)] = cnt_tot
    stage_v[pl.ds(L, L)] = conf_tot
    stage_v[pl.ds(2 * L, L)] = acc_tot
    pltpu.sync_copy(stage_v, out_hbm.at[wid])

  def _tc_hist_body(conf_ref, tgt_ref, out_ref):
    g = pl.program_id(0)

    @pl.when(g == 0)
    def _():
      for i in range(48):
        out_ref[i] = jnp.float32(0.0)

    c = conf_ref[...]
    t = tgt_ref[...]
    accur = ((c >= 0.5) == (t == 1)).astype(jnp.float32)
    for i in range(N_BINS):
      m = c <= float(_BOUNDS_F32[i])
      out_ref[i] += jnp.sum(m.astype(jnp.float32))
      out_ref[16 + i] += jnp.sum(jnp.where(m, c, 0.0))
      out_ref[32 + i] += jnp.sum(jnp.where(m, accur, 0.0))
    out_ref[31] += jnp.sum(c)
    out_ref[47] += jnp.sum(accur)

  def _tc_hist_kernel(conf2d, tgt2d):
    return pl.pallas_call(
        _tc_hist_body,
        grid=(TC_STEPS,),
        in_specs=[
            pl.BlockSpec((TC_BR, TC_COLS),
                         lambda g: (TC_ROW0 // TC_BR + g, 0)),
            pl.BlockSpec((TC_BR, TC_COLS),
                         lambda g: (TC_ROW0 // TC_BR + g, 0)),
        ],
        out_specs=pl.BlockSpec(memory_space=pltpu.SMEM),
        out_shape=jax.ShapeDtypeStruct((48,), jnp.float32),
    )(conf2d, tgt2d)

  def _finalize_body(part_ref, tcv_ref, out_ref):
    s = jnp.sum(part_ref[...], axis=0)
    tc = tcv_ref[...]
    def bins(le15, total):
      hi = jnp.concatenate([le15[1:], total[None]])
      return hi - le15
    tc_cnt = bins(tc[0:15], jnp.float32(TC_ELEMS))
    tc_conf = bins(tc[16:31], tc[31])
    tc_acc = bins(tc[32:47], tc[47])
    cnt = s[0:15] + tc_cnt
    conf = s[16:31] + tc_conf
    acc = s[32:47] + tc_acc
    safe = jnp.maximum(cnt, 1.0)
    contrib = jnp.abs(conf / safe - acc / safe) * (cnt * (1.0 / N_ELEMS))
    contrib = jnp.where(cnt > 0.0, contrib, 0.0)
    out_ref[...] = jnp.full((1,), jnp.sum(contrib), jnp.float32)

  def _finalize_kernel(partials, tcvec):
    return pl.pallas_call(
        _finalize_body,
        out_shape=jax.ShapeDtypeStruct((1,), jnp.float32),
    )(partials, tcvec)

  return _hist_kernel, _tc_hist_kernel, _finalize_kernel


def kernel(inputs, targets):
  hist_kernel, tc_hist_kernel, finalize_kernel = _build_kernels()
  btab = jnp.asarray(_BTAB)
  partials = hist_kernel(inputs, targets, btab)
  conf2d = inputs.reshape(N_ELEMS // TC_COLS, TC_COLS)
  tgt2d = targets.reshape(N_ELEMS // TC_COLS, TC_COLS)
  tcvec = tc_hist_kernel(conf2d, tgt2d)
  return finalize_kernel(partials, tcvec)
